# baseline (speedup 1.0000x reference)
import functools
import math

import jax
import jax.numpy as jnp
from jax import lax
from jax.experimental import pallas as pl
from jax.experimental.pallas import tpu as pltpu
from jax.experimental.pallas import tpu_sc as plsc

N = 10000
E = 320000
HIDDEN = 128
NUM_LAYERS = 4
ALPHA = 0.1
THETA = 0.5
NUM_GRAPHS = 32

NC = 2
NS = 16
NW = NC * NS
CHUNK = 64
NCHT = E // CHUNK
CPW = NCHT // NW
NEXTRA = NCHT - CPW * NW
NSLOT = 6
NBLK = CPW // NSLOT
ROWS_PER_TILE = 624
ROWS_REM = N - NS * ROWS_PER_TILE


def _sc_segment_sum(h, src, dst, zeros):
    mesh = plsc.VectorSubcoreMesh(core_axis_name="c", subcore_axis_name="s")

    @functools.partial(
        pl.kernel,
        out_type=jax.ShapeDtypeStruct((NC * N, HIDDEN), jnp.float32),
        mesh=mesh,
        scratch_types=(
            [pltpu.VMEM((CHUNK,), jnp.int32)] * NSLOT
            + [pltpu.VMEM((CHUNK,), jnp.int32)] * NSLOT
            + [pltpu.VMEM((CHUNK, HIDDEN), jnp.float32)] * NSLOT
            + [pltpu.VMEM_SHARED((N, HIDDEN), jnp.float32)]
            + [pltpu.SemaphoreType.DMA] * (4 * NSLOT + 1)
        ),
    )
    def k(h_hbm, src_hbm, dst_hbm, z_hbm, out_hbm, *refs):
        sis = refs[0:NSLOT]
        dis = refs[NSLOT:2 * NSLOT]
        rbs = refs[2 * NSLOT:3 * NSLOT]
        agg = refs[3 * NSLOT]
        gss = refs[3 * NSLOT + 1:3 * NSLOT + 1 + NSLOT]
        sss = refs[3 * NSLOT + 1 + NSLOT:3 * NSLOT + 1 + 2 * NSLOT]
        isg = refs[3 * NSLOT + 1 + 2 * NSLOT:3 * NSLOT + 1 + 3 * NSLOT]
        isd = refs[3 * NSLOT + 1 + 3 * NSLOT:3 * NSLOT + 1 + 4 * NSLOT]
        zs = refs[-1]
        c = lax.axis_index("c")
        s = lax.axis_index("s")
        wid = c * NS + s
        row0 = s * ROWS_PER_TILE
        cw0 = wid * CPW + jnp.minimum(wid, NEXTRA)
        ncw = CPW + jnp.where(wid < NEXTRA, 1, 0)

        def si_src(ci):
            return src_hbm.at[pl.ds((cw0 + ci) * CHUNK, CHUNK)]

        def di_src(ci):
            return dst_hbm.at[pl.ds((cw0 + ci) * CHUNK, CHUNK)]

        def g_issue(ci, t):
            pltpu.async_copy(h_hbm.at[sis[t]], rbs[t], gss[t])

        def g_wait(t):
            pltpu.make_async_copy(h_hbm.at[sis[t]], rbs[t], gss[t]).wait()

        def s_issue(t):
            pltpu.async_copy(rbs[t], agg.at[dis[t]], sss[t], add=True)

        def s_wait(t):
            pltpu.make_async_copy(rbs[t], agg.at[dis[t]], sss[t]).wait()

        zc = pltpu.async_copy(z_hbm.at[pl.ds(row0, ROWS_PER_TILE)],
                              agg.at[pl.ds(row0, ROWS_PER_TILE)], zs)

        @pl.when(s == NS - 1)
        def _():
            pltpu.sync_copy(z_hbm.at[pl.ds(NS * ROWS_PER_TILE, ROWS_REM)],
                            agg.at[pl.ds(NS * ROWS_PER_TILE, ROWS_REM)])

        for t in range(NSLOT):
            pltpu.async_copy(si_src(t), sis[t], isg[t])
        for t in range(3):
            pltpu.async_copy(di_src(t), dis[t], isd[t])
        for t in range(3):
            pltpu.make_async_copy(si_src(t), sis[t], isg[t]).wait()
            g_issue(t, t)
        zc.wait()
        plsc.subcore_barrier()

        def body(j, carry):
            c0 = j * NSLOT
            for t in range(3):
                ci = c0 + t
                g_wait(t)

                @pl.when(ci + NSLOT < ncw)
                def _():
                    pltpu.async_copy(si_src(ci + NSLOT), sis[t], isg[t])

                pltpu.make_async_copy(di_src(ci), dis[t], isd[t]).wait()
                s_issue(t)
            for t in range(3, NSLOT):
                cg = c0 + t

                @pl.when(j > 0)
                def _():
                    s_wait(t)

                pltpu.async_copy(di_src(cg), dis[t], isd[t])
                pltpu.make_async_copy(si_src(cg), sis[t], isg[t]).wait()
                g_issue(cg, t)
            for t in range(3, NSLOT):
                ci = c0 + t
                g_wait(t)

                @pl.when(ci + NSLOT < ncw)
                def _():
                    pltpu.async_copy(si_src(ci + NSLOT), sis[t], isg[t])

                pltpu.make_async_copy(di_src(ci), dis[t], isd[t]).wait()
                s_issue(t)
            for t in range(3):
                cg = c0 + NSLOT + t
                s_wait(t)

                @pl.when(cg < CPW)
                def _():
                    pltpu.async_copy(di_src(cg), dis[t], isd[t])
                    pltpu.make_async_copy(si_src(cg), sis[t], isg[t]).wait()
                    g_issue(cg, t)

            return carry

        lax.fori_loop(0, NBLK // 2,
                      lambda j, cr: body(2 * j + 1, body(2 * j, cr)), 0)

        for t in range(3, NSLOT):
            s_wait(t)

        @pl.when(wid < NEXTRA)
        def _():
            pltpu.make_async_copy(si_src(CPW), sis[0], isg[0]).wait()
            pltpu.sync_copy(di_src(CPW), dis[0])
            pltpu.async_copy(h_hbm.at[sis[0]], rbs[0], gss[0]).wait()
            pltpu.sync_copy(rbs[0], agg.at[dis[0]], add=True)

        plsc.subcore_barrier()

        obase = c * N
        pltpu.sync_copy(agg.at[pl.ds(row0, ROWS_PER_TILE)],
                        out_hbm.at[pl.ds(obase + row0, ROWS_PER_TILE)])

        @pl.when(s == NS - 1)
        def _():
            pltpu.sync_copy(agg.at[pl.ds(NS * ROWS_PER_TILE, ROWS_REM)],
                            out_hbm.at[pl.ds(obase + NS * ROWS_PER_TILE, ROWS_REM)])

    return k(h, src, dst, zeros)


def _tc_lin0(x, W0, b0):
    def body(x_ref, w_ref, b_ref, o_ref):
        o_ref[...] = jnp.maximum(
            jnp.dot(x_ref[...], w_ref[...],
                    preferred_element_type=jnp.float32) + b_ref[...], 0.0)

    return pl.pallas_call(
        body,
        out_shape=jax.ShapeDtypeStruct((N, HIDDEN), jnp.float32),
    )(x, W0, b0.reshape(1, HIDDEN))


def _tc_layer(p, h, x0, W, beta):
    def body(p_ref, h_ref, x0_ref, w_ref, o_ref):
        agg = p_ref[:N, :] + p_ref[N:, :]
        out = agg * (1.0 - ALPHA) + ALPHA * x0_ref[...]
        out = (1.0 - beta) * out + beta * jnp.dot(
            out, w_ref[...], preferred_element_type=jnp.float32)
        h2 = jnp.maximum(out + h_ref[...], 0.0)
        mu = jnp.mean(h2, axis=0, keepdims=True)
        hm = h2 - mu
        denom = jnp.sqrt(1e-5 + jnp.sum(hm * hm) / N)
        o_ref[...] = hm / denom

    return pl.pallas_call(
        body,
        out_shape=jax.ShapeDtypeStruct((N, HIDDEN), jnp.float32),
    )(p, h, x0, W)


def _tc_layer_pool(p, h, x0, W, beta, batch2d, W1, b1, W2, b2, gamma, bnb):
    def body(p_ref, h_ref, x0_ref, w_ref, b_ref, w1_ref, b1_ref, w2_ref,
             b2_ref, g_ref, bb_ref, o_ref):
        agg = p_ref[:N, :] + p_ref[N:, :]
        out = agg * (1.0 - ALPHA) + ALPHA * x0_ref[...]
        out = (1.0 - beta) * out + beta * jnp.dot(
            out, w_ref[...], preferred_element_type=jnp.float32)
        h2 = jnp.maximum(out + h_ref[...], 0.0)
        mu = jnp.mean(h2, axis=0, keepdims=True)
        hm = h2 - mu
        hv = hm / jnp.sqrt(1e-5 + jnp.sum(hm * hm) / N)

        bids = b_ref[...]
        gids = lax.broadcasted_iota(jnp.int32, (1, NUM_GRAPHS), 1)
        maskf = (bids == gids).astype(jnp.float32)
        counts = jnp.sum(maskf, axis=0)
        gsum = lax.dot_general(maskf, hv, (((0,), (0,)), ((), ())),
                               preferred_element_type=jnp.float32)
        gmean = gsum / jnp.maximum(counts, 1.0)[:, None]
        cols = []
        for g in range(NUM_GRAPHS):
            m = maskf[:, g:g + 1] > 0
            cols.append(jnp.max(jnp.where(m, hv, -3.4e38), axis=0,
                                keepdims=True))
        gmax = jnp.concatenate(cols, axis=0)
        x2 = jnp.concatenate([gmax, gmean], axis=1)
        z = jnp.maximum(
            jnp.dot(x2, w1_ref[...], preferred_element_type=jnp.float32)
            + b1_ref[...], 0.0)
        z = g_ref[...] * z / jnp.sqrt(1.0 + 1e-5) + bb_ref[...]
        z = jnp.dot(z, w2_ref[...], preferred_element_type=jnp.float32) \
            + b2_ref[...]
        zm = jnp.max(z, axis=1, keepdims=True)
        ze = z - zm
        lse = jnp.log(jnp.sum(jnp.exp(ze), axis=1, keepdims=True))
        o_ref[...] = ze - lse

    return pl.pallas_call(
        body,
        out_shape=jax.ShapeDtypeStruct((NUM_GRAPHS, 2), jnp.float32),
    )(p, h, x0, W, batch2d, W1, b1, W2, b2, gamma, bnb)


def kernel(x, edge_index, batch, W0, b0, convW, W1, b1, W2, b2,
           bn_gamma, bn_beta):
    src = edge_index[0]
    dst = edge_index[1]
    zeros = jnp.zeros((N, HIDDEN), jnp.float32)
    h = _tc_lin0(x, W0, b0)
    x0 = h
    for l in range(NUM_LAYERS - 1):
        beta = float(math.log(THETA / (l + 1) + 1.0))
        p = _sc_segment_sum(h, src, dst, zeros)
        h = _tc_layer(p, h, x0, convW[l], beta)
    beta = float(math.log(THETA / NUM_LAYERS + 1.0))
    p = _sc_segment_sum(h, src, dst, zeros)
    return _tc_layer_pool(p, h, x0, convW[NUM_LAYERS - 1], beta,
                          batch.reshape(N, 1), W1, b1.reshape(1, -1),
                          W2, b2.reshape(1, -1), bn_gamma.reshape(1, -1),
                          bn_beta.reshape(1, -1))


# speedup vs baseline: 1.0019x; 1.0019x over previous
import functools
import math

import jax
import jax.numpy as jnp
from jax import lax
from jax.experimental import pallas as pl
from jax.experimental.pallas import tpu as pltpu
from jax.experimental.pallas import tpu_sc as plsc

N = 10000
E = 320000
HIDDEN = 128
NUM_LAYERS = 4
ALPHA = 0.1
THETA = 0.5
NUM_GRAPHS = 32

NC = 2
NS = 16
NW = NC * NS
CHUNK = 64
NCHT = E // CHUNK
CPW = NCHT // NW
NEXTRA = NCHT - CPW * NW
NSLOT = 6
NBLK = CPW // NSLOT
ROWS_PER_TILE = 624
ROWS_REM = N - NS * ROWS_PER_TILE


def _sc_segment_sum(h, src, dst, zeros):
    mesh = plsc.VectorSubcoreMesh(core_axis_name="c", subcore_axis_name="s")

    @functools.partial(
        pl.kernel,
        out_type=jax.ShapeDtypeStruct((NC * N, HIDDEN), jnp.float32),
        mesh=mesh,
        scratch_types=(
            [pltpu.VMEM((CHUNK,), jnp.int32)] * NSLOT
            + [pltpu.VMEM((CHUNK,), jnp.int32)] * NSLOT
            + [pltpu.VMEM((CHUNK, HIDDEN), jnp.float32)] * NSLOT
            + [pltpu.VMEM_SHARED((N, HIDDEN), jnp.float32)]
            + [pltpu.SemaphoreType.DMA] * (4 * NSLOT + 1)
        ),
    )
    def k(h_hbm, src_hbm, dst_hbm, z_hbm, out_hbm, *refs):
        sis = refs[0:NSLOT]
        dis = refs[NSLOT:2 * NSLOT]
        rbs = refs[2 * NSLOT:3 * NSLOT]
        agg = refs[3 * NSLOT]
        gss = refs[3 * NSLOT + 1:3 * NSLOT + 1 + NSLOT]
        sss = refs[3 * NSLOT + 1 + NSLOT:3 * NSLOT + 1 + 2 * NSLOT]
        isg = refs[3 * NSLOT + 1 + 2 * NSLOT:3 * NSLOT + 1 + 3 * NSLOT]
        isd = refs[3 * NSLOT + 1 + 3 * NSLOT:3 * NSLOT + 1 + 4 * NSLOT]
        zs = refs[-1]
        c = lax.axis_index("c")
        s = lax.axis_index("s")
        wid = c * NS + s
        row0 = s * ROWS_PER_TILE
        cw0 = wid * CPW + jnp.minimum(wid, NEXTRA)
        ncw = CPW + jnp.where(wid < NEXTRA, 1, 0)

        def si_src(ci):
            return src_hbm.at[pl.ds((cw0 + ci) * CHUNK, CHUNK)]

        def di_src(ci):
            return dst_hbm.at[pl.ds((cw0 + ci) * CHUNK, CHUNK)]

        def g_issue(ci, t):
            pltpu.async_copy(h_hbm.at[sis[t]], rbs[t], gss[t])

        def g_wait(t):
            pltpu.make_async_copy(h_hbm.at[sis[t]], rbs[t], gss[t]).wait()

        def s_issue(t):
            pltpu.async_copy(rbs[t], agg.at[dis[t]], sss[t], add=True)

        def s_wait(t):
            pltpu.make_async_copy(rbs[t], agg.at[dis[t]], sss[t]).wait()

        zc = pltpu.async_copy(z_hbm.at[pl.ds(row0, ROWS_PER_TILE)],
                              agg.at[pl.ds(row0, ROWS_PER_TILE)], zs)

        @pl.when(s == NS - 1)
        def _():
            pltpu.sync_copy(z_hbm.at[pl.ds(NS * ROWS_PER_TILE, ROWS_REM)],
                            agg.at[pl.ds(NS * ROWS_PER_TILE, ROWS_REM)])

        for t in range(NSLOT):
            pltpu.async_copy(si_src(t), sis[t], isg[t])
        for t in range(3):
            pltpu.async_copy(di_src(t), dis[t], isd[t])
        for t in range(3):
            pltpu.make_async_copy(si_src(t), sis[t], isg[t]).wait()
            g_issue(t, t)
        zc.wait()
        plsc.subcore_barrier()

        def body(j, carry):
            c0 = j * NSLOT
            for t in range(3):
                ci = c0 + t
                g_wait(t)

                @pl.when(ci + NSLOT < ncw)
                def _():
                    pltpu.async_copy(si_src(ci + NSLOT), sis[t], isg[t])

                pltpu.make_async_copy(di_src(ci), dis[t], isd[t]).wait()
                s_issue(t)
            for t in range(3, NSLOT):
                cg = c0 + t

                @pl.when(j > 0)
                def _():
                    s_wait(t)

                pltpu.async_copy(di_src(cg), dis[t], isd[t])
                pltpu.make_async_copy(si_src(cg), sis[t], isg[t]).wait()
                g_issue(cg, t)
            for t in range(3, NSLOT):
                ci = c0 + t
                g_wait(t)

                @pl.when(ci + NSLOT < ncw)
                def _():
                    pltpu.async_copy(si_src(ci + NSLOT), sis[t], isg[t])

                pltpu.make_async_copy(di_src(ci), dis[t], isd[t]).wait()
                s_issue(t)
            for t in range(3):
                cg = c0 + NSLOT + t
                s_wait(t)

                @pl.when(cg < CPW)
                def _():
                    pltpu.async_copy(di_src(cg), dis[t], isd[t])
                    pltpu.make_async_copy(si_src(cg), sis[t], isg[t]).wait()
                    g_issue(cg, t)

            return carry

        lax.fori_loop(0, NBLK, body, 0)

        for t in range(3, NSLOT):
            s_wait(t)

        @pl.when(wid < NEXTRA)
        def _():
            pltpu.make_async_copy(si_src(CPW), sis[0], isg[0]).wait()
            pltpu.sync_copy(di_src(CPW), dis[0])
            pltpu.async_copy(h_hbm.at[sis[0]], rbs[0], gss[0]).wait()
            pltpu.sync_copy(rbs[0], agg.at[dis[0]], add=True)

        plsc.subcore_barrier()

        obase = c * N
        pltpu.sync_copy(agg.at[pl.ds(row0, ROWS_PER_TILE)],
                        out_hbm.at[pl.ds(obase + row0, ROWS_PER_TILE)])

        @pl.when(s == NS - 1)
        def _():
            pltpu.sync_copy(agg.at[pl.ds(NS * ROWS_PER_TILE, ROWS_REM)],
                            out_hbm.at[pl.ds(obase + NS * ROWS_PER_TILE, ROWS_REM)])

    return k(h, src, dst, zeros)


def _tc_lin0(x, W0, b0):
    def body(x_ref, w_ref, b_ref, o_ref):
        o_ref[...] = jnp.maximum(
            jnp.dot(x_ref[...], w_ref[...],
                    preferred_element_type=jnp.float32) + b_ref[...], 0.0)

    return pl.pallas_call(
        body,
        out_shape=jax.ShapeDtypeStruct((N, HIDDEN), jnp.float32),
    )(x, W0, b0.reshape(1, HIDDEN))


def _tc_layer(p, h, x0, W, beta):
    def body(p_ref, h_ref, x0_ref, w_ref, o_ref):
        agg = p_ref[:N, :] + p_ref[N:, :]
        out = agg * (1.0 - ALPHA) + ALPHA * x0_ref[...]
        out = (1.0 - beta) * out + beta * jnp.dot(
            out, w_ref[...], preferred_element_type=jnp.float32)
        h2 = jnp.maximum(out + h_ref[...], 0.0)
        mu = jnp.mean(h2, axis=0, keepdims=True)
        hm = h2 - mu
        denom = jnp.sqrt(1e-5 + jnp.sum(hm * hm) / N)
        o_ref[...] = hm / denom

    return pl.pallas_call(
        body,
        out_shape=jax.ShapeDtypeStruct((N, HIDDEN), jnp.float32),
    )(p, h, x0, W)


def _tc_layer_pool(p, h, x0, W, beta, batch2d, W1, b1, W2, b2, gamma, bnb):
    def body(p_ref, h_ref, x0_ref, w_ref, b_ref, w1_ref, b1_ref, w2_ref,
             b2_ref, g_ref, bb_ref, o_ref):
        agg = p_ref[:N, :] + p_ref[N:, :]
        out = agg * (1.0 - ALPHA) + ALPHA * x0_ref[...]
        out = (1.0 - beta) * out + beta * jnp.dot(
            out, w_ref[...], preferred_element_type=jnp.float32)
        h2 = jnp.maximum(out + h_ref[...], 0.0)
        mu = jnp.mean(h2, axis=0, keepdims=True)
        hm = h2 - mu
        hv = hm / jnp.sqrt(1e-5 + jnp.sum(hm * hm) / N)

        bids = b_ref[...]
        gids = lax.broadcasted_iota(jnp.int32, (1, NUM_GRAPHS), 1)
        maskf = (bids == gids).astype(jnp.float32)
        counts = jnp.sum(maskf, axis=0)
        gsum = lax.dot_general(maskf, hv, (((0,), (0,)), ((), ())),
                               preferred_element_type=jnp.float32)
        gmean = gsum / jnp.maximum(counts, 1.0)[:, None]
        cols = []
        for g in range(NUM_GRAPHS):
            m = maskf[:, g:g + 1] > 0
            cols.append(jnp.max(jnp.where(m, hv, -3.4e38), axis=0,
                                keepdims=True))
        gmax = jnp.concatenate(cols, axis=0)
        x2 = jnp.concatenate([gmax, gmean], axis=1)
        z = jnp.maximum(
            jnp.dot(x2, w1_ref[...], preferred_element_type=jnp.float32)
            + b1_ref[...], 0.0)
        z = g_ref[...] * z / jnp.sqrt(1.0 + 1e-5) + bb_ref[...]
        z = jnp.dot(z, w2_ref[...], preferred_element_type=jnp.float32) \
            + b2_ref[...]
        zm = jnp.max(z, axis=1, keepdims=True)
        ze = z - zm
        lse = jnp.log(jnp.sum(jnp.exp(ze), axis=1, keepdims=True))
        o_ref[...] = ze - lse

    return pl.pallas_call(
        body,
        out_shape=jax.ShapeDtypeStruct((NUM_GRAPHS, 2), jnp.float32),
    )(p, h, x0, W, batch2d, W1, b1, W2, b2, gamma, bnb)


def kernel(x, edge_index, batch, W0, b0, convW, W1, b1, W2, b2,
           bn_gamma, bn_beta):
    src = edge_index[0]
    dst = edge_index[1]
    zeros = jnp.zeros((N, HIDDEN), jnp.float32)
    h = _tc_lin0(x, W0, b0)
    x0 = h
    for l in range(NUM_LAYERS - 1):
        beta = float(math.log(THETA / (l + 1) + 1.0))
        p = _sc_segment_sum(h, src, dst, zeros)
        h = _tc_layer(p, h, x0, convW[l], beta)
    beta = float(math.log(THETA / NUM_LAYERS + 1.0))
    p = _sc_segment_sum(h, src, dst, zeros)
    return _tc_layer_pool(p, h, x0, convW[NUM_LAYERS - 1], beta,
                          batch.reshape(N, 1), W1, b1.reshape(1, -1),
                          W2, b2.reshape(1, -1), bn_gamma.reshape(1, -1),
                          bn_beta.reshape(1, -1))
